# Initial kernel scaffold; baseline (speedup 1.0000x reference)
import jax
import jax.numpy as jnp
from jax.experimental import pallas as pl


def kernel(x, edge_index, W1, a1, W2, a2, W3, a3):
    raise NotImplementedError("write your pallas kernel here")


# speedup vs baseline: 32.6899x; 32.6899x over previous
import functools

import jax
import jax.numpy as jnp
from jax import lax
from jax.experimental import pallas as pl
from jax.experimental.pallas import tpu as pltpu
from jax.experimental.pallas import tpu_sc as plsc

N = 10000
E = 320000
D_IN, H1, H2 = 128, 64, 32

NC, NS = 2, 16
R_E = E // 128
RPT = 80
R_P = NC * NS * RPT
NP = 10240
NPS = NP // NS
BLK = 1024
GRID = NP // BLK

F32 = jnp.float32


def _tc1_body(x_ref, w_ref, a_ref, wh_ref, s_ref):
    wh = jnp.dot(x_ref[...], w_ref[...], preferred_element_type=F32)
    wh_ref[...] = wh
    a = a_ref[...]
    sa = jnp.dot(wh, a[:H1, :], preferred_element_type=F32)
    sb = jnp.dot(wh, a[H1:, :], preferred_element_type=F32)
    s_ref[...] = jnp.concatenate([sa, sb], axis=1)


def _tc1(x, W1, a1):
    return pl.pallas_call(
        _tc1_body,
        grid=(GRID,),
        in_specs=[
            pl.BlockSpec((BLK, D_IN), lambda i: (i, 0)),
            pl.BlockSpec((D_IN, H1), lambda i: (0, 0)),
            pl.BlockSpec((2 * H1, 1), lambda i: (0, 0)),
        ],
        out_specs=[
            pl.BlockSpec((BLK, H1), lambda i: (i, 0)),
            pl.BlockSpec((BLK, 2), lambda i: (i, 0)),
        ],
        out_shape=[
            jax.ShapeDtypeStruct((NP, H1), F32),
            jax.ShapeDtypeStruct((NP, 2), F32),
        ],
    )(x, W1, a1)


def _tc2_body(hp_ref, w2_ref, w3_ref, a2_ref, a3_ref, wh_ref, s_ref):
    h = jnp.maximum(hp_ref[0] + hp_ref[1], 0.0)
    w23 = jnp.concatenate([w2_ref[...], w3_ref[...]], axis=1)
    wh = jnp.dot(h, w23, preferred_element_type=F32)
    wh_ref[...] = wh
    a2 = a2_ref[...]
    a3 = a3_ref[...]
    s2a = jnp.dot(wh[:, :H2], a2[:H2, :], preferred_element_type=F32)
    s2b = jnp.dot(wh[:, :H2], a2[H2:, :], preferred_element_type=F32)
    s3a = jnp.dot(wh[:, H2:], a3[:H2, :], preferred_element_type=F32)
    s3b = jnp.dot(wh[:, H2:], a3[H2:, :], preferred_element_type=F32)
    s_ref[...] = jnp.concatenate([s2a, s2b, s3a, s3b], axis=1)


def _tc2(hp1, W2, W3, a2, a3):
    return pl.pallas_call(
        _tc2_body,
        grid=(GRID,),
        in_specs=[
            pl.BlockSpec((NC, BLK, H1), lambda i: (0, i, 0)),
            pl.BlockSpec((H1, H2), lambda i: (0, 0)),
            pl.BlockSpec((H1, H2), lambda i: (0, 0)),
            pl.BlockSpec((2 * H2, 1), lambda i: (0, 0)),
            pl.BlockSpec((2 * H2, 1), lambda i: (0, 0)),
        ],
        out_specs=[
            pl.BlockSpec((BLK, 2 * H2), lambda i: (i, 0)),
            pl.BlockSpec((BLK, 4), lambda i: (i, 0)),
        ],
        out_shape=[
            jax.ShapeDtypeStruct((NP, 2 * H2), F32),
            jax.ShapeDtypeStruct((NP, 4), F32),
        ],
    )(hp1, W2, W3, a2, a3)


def _tc3_body(hp_ref, mu_ref, lv_ref):
    m = hp_ref[0] + hp_ref[1]
    mu_ref[...] = m[:, :H2]
    lv_ref[...] = m[:, H2:]


def _tc3(hp23):
    return pl.pallas_call(
        _tc3_body,
        grid=(GRID,),
        in_specs=[pl.BlockSpec((NC, BLK, 2 * H2), lambda i: (0, i, 0))],
        out_specs=[
            pl.BlockSpec((BLK, H2), lambda i: (i, 0)),
            pl.BlockSpec((BLK, H2), lambda i: (i, 0)),
        ],
        out_shape=[
            jax.ShapeDtypeStruct((NP, H2), F32),
            jax.ShapeDtypeStruct((NP, H2), F32),
        ],
    )(hp23)


def _make_sc(nch):
    hw = 64
    st = 2 * nch * N
    oth_blk = 40
    mesh = plsc.VectorSubcoreMesh(core_axis_name="c", subcore_axis_name="s")

    scratch = [
        pltpu.VMEM((st,), F32),
        pltpu.VMEM((RPT + oth_blk, 128), jnp.int32),
        pltpu.VMEM((RPT + oth_blk, 128), jnp.int32),
        pltpu.VMEM((nch, 128), F32),
        pltpu.VMEM((1, 128), jnp.int32),
        pltpu.VMEM((nch * NPS,), F32),
        pltpu.VMEM((128, hw), F32),
        pltpu.VMEM_SHARED((nch * NP,), F32),
        pltpu.VMEM_SHARED((NP, hw), F32),
        pltpu.SemaphoreType.DMA,
    ]

    @functools.partial(
        pl.kernel,
        out_type=jax.ShapeDtypeStruct((NC, NP, hw), F32),
        mesh=mesh,
        scratch_types=scratch,
        compiler_params=pltpu.CompilerParams(use_tc_tiling_on_sc=False,
                                             needs_layout_passes=False),
    )
    def sc_kernel(src_hbm, dst_hbm, s_hbm, wh_hbm, hp_out,
                  s_v, src_v, dst_v, exr_v, idx1_v, den_v, rows_v,
                  den_sh, hp_sh, sem):
        cid = lax.axis_index("c")
        sid = lax.axis_index("s")

        own = (cid * NS + sid) * RPT
        oth = ((1 - cid) * NS + sid) * RPT
        pltpu.sync_copy(s_hbm, s_v)
        pltpu.sync_copy(src_hbm.at[pl.ds(own, RPT)], src_v.at[pl.ds(0, RPT)])
        pltpu.sync_copy(dst_hbm.at[pl.ds(own, RPT)], dst_v.at[pl.ds(0, RPT)])

        def zrow(j, c):
            for q in range(hw // 16):
                rows_v[j, pl.ds(q * 16, 16)] = jnp.zeros((16,), F32)
            return c
        lax.fori_loop(0, 128, zrow, 0)

        def zden(k, c):
            den_v[pl.ds(k * 16, 16)] = jnp.zeros((16,), F32)
            return c
        lax.fori_loop(0, nch * NPS // 16, zden, 0)
        for ch in range(nch):
            pltpu.sync_copy(den_v.at[pl.ds(ch * NPS, NPS)],
                            den_sh.at[pl.ds(ch * NP + sid * NPS, NPS)])
        for o in range(0, NPS, 128):
            pltpu.sync_copy(rows_v, hp_sh.at[pl.ds(sid * NPS + o, 128)])
        plsc.subcore_barrier()

        def compute_ex(r):
            for k in range(8):
                sl = pl.ds(k * 16, 16)
                s16 = src_v[r, sl]
                d16 = dst_v[r, sl]
                if nch == 1:
                    g0 = plsc.load_gather(s_v, [s16 * 2])
                    g1 = plsc.load_gather(s_v, [d16 * 2 + 1])
                    l0 = g0 + g1
                    l0 = jnp.where(l0 >= 0.0, l0, l0 * 0.01) * 2.0
                    exr_v[0, sl] = jnp.exp(jnp.minimum(l0, 80.0))
                else:
                    g0 = plsc.load_gather(s_v, [s16 * 4])
                    g1 = plsc.load_gather(s_v, [d16 * 4 + 1])
                    g2 = plsc.load_gather(s_v, [s16 * 4 + 2])
                    g3 = plsc.load_gather(s_v, [d16 * 4 + 3])
                    l0 = g0 + g1
                    l0 = jnp.where(l0 >= 0.0, l0, l0 * 0.01) * 2.0
                    l1 = g2 + g3
                    l1 = jnp.where(l1 >= 0.0, l1, l1 * 0.01) * 2.0
                    exr_v[0, sl] = jnp.exp(jnp.minimum(l0, 80.0))
                    exr_v[1, sl] = jnp.exp(jnp.minimum(l1, 80.0))
                    idx1_v[0, sl] = s16 + NP

        def den_scatter(r):
            pltpu.sync_copy(exr_v.at[0], den_sh.at[src_v.at[r]], add=True)
            if nch == 2:
                pltpu.sync_copy(exr_v.at[1], den_sh.at[idx1_v.at[0]], add=True)

        n_own = jnp.clip(R_E - own, 0, RPT)
        n_oth = jnp.clip(R_E - oth, 0, RPT)

        def own_body(r, c):
            compute_ex(r)
            den_scatter(r)
            pltpu.async_copy(wh_hbm.at[dst_v.at[r]], rows_v, sem).wait()
            for k in range(8):
                sl = pl.ds(k * 16, 16)
                e0 = exr_v[0, sl]
                e1 = exr_v[1, sl] if nch == 2 else e0
                for jj in range(16):
                    j = k * 16 + jj
                    a0 = e0[jj]
                    a1 = e1[jj]
                    for q in range(2):
                        qs = pl.ds(q * 16, 16)
                        rows_v[j, qs] = rows_v[j, qs] * a0
                    for q in range(2, 4):
                        qs = pl.ds(q * 16, 16)
                        rows_v[j, qs] = rows_v[j, qs] * a1
            pltpu.sync_copy(rows_v, hp_sh.at[src_v.at[r]], add=True)
            return c

        def oth_body(r, c):
            compute_ex(RPT + r)
            den_scatter(RPT + r)
            return c

        lax.fori_loop(0, n_own, own_body, 0)
        for half in range(RPT // oth_blk):
            pltpu.sync_copy(src_hbm.at[pl.ds(oth + half * oth_blk, oth_blk)],
                            src_v.at[pl.ds(RPT, oth_blk)])
            pltpu.sync_copy(dst_hbm.at[pl.ds(oth + half * oth_blk, oth_blk)],
                            dst_v.at[pl.ds(RPT, oth_blk)])
            cnt = jnp.clip(n_oth - half * oth_blk, 0, oth_blk)
            lax.fori_loop(0, cnt, oth_body, 0)
        plsc.subcore_barrier()

        for ch in range(nch):
            pltpu.sync_copy(den_sh.at[pl.ds(ch * NP + sid * NPS, NPS)],
                            den_v.at[pl.ds(ch * NPS, NPS)])
        base = sid * NPS

        def norm_body(b, c):
            pltpu.async_copy(hp_sh.at[pl.ds(base + b * 128, 128)], rows_v,
                             sem).wait()
            for k in range(8):
                sl = pl.ds(b * 128 + k * 16, 16)
                d0 = den_v[sl]
                r0 = 1.0 / jnp.maximum(d0, 1e-30)
                if nch == 2:
                    d1 = den_v[pl.ds(NPS + b * 128 + k * 16, 16)]
                    r1 = 1.0 / jnp.maximum(d1, 1e-30)
                else:
                    r1 = r0
                for jj in range(16):
                    j = k * 16 + jj
                    a0 = r0[jj]
                    a1 = r1[jj]
                    for q in range(2):
                        qs = pl.ds(q * 16, 16)
                        rows_v[j, qs] = rows_v[j, qs] * a0
                    for q in range(2, 4):
                        qs = pl.ds(q * 16, 16)
                        rows_v[j, qs] = rows_v[j, qs] * a1
            pltpu.sync_copy(rows_v,
                            hp_out.at[cid, pl.ds(base + b * 128, 128)])
            return c

        lax.fori_loop(0, NPS // 128, norm_body, 0)

    return sc_kernel


_sc1 = _make_sc(1)
_sc23 = _make_sc(2)


def kernel(x, edge_index, W1, a1, W2, a2, W3, a3):
    src2d = jnp.pad(edge_index[0].reshape(R_E, 128), ((0, R_P - R_E), (0, 0)))
    dst2d = jnp.pad(edge_index[1].reshape(R_E, 128), ((0, R_P - R_E), (0, 0)))
    x_p = jnp.pad(x, ((0, NP - N), (0, 0)))

    wh1, s1 = _tc1(x_p, W1, a1)
    hp1 = _sc1(src2d, dst2d, s1.reshape(-1)[:2 * N], wh1)
    wh23, s23 = _tc2(hp1, W2, W3, a2, a3)
    hp23 = _sc23(src2d, dst2d, s23.reshape(-1)[:4 * N], wh23)
    mu, logvar = _tc3(hp23)
    return (mu[:N], mu[:N], logvar[:N])


# speedup vs baseline: 42.1593x; 1.2897x over previous
import functools

import jax
import jax.numpy as jnp
from jax import lax
from jax.experimental import pallas as pl
from jax.experimental.pallas import tpu as pltpu
from jax.experimental.pallas import tpu_sc as plsc

N = 10000
E = 320000
D_IN, H1, H2 = 128, 64, 32

NC, NS = 2, 16
R_E = E // 128
RPT = 80
R_P = NC * NS * RPT
NP = 10240
NPS = NP // NS
BLK = 1024
GRID = NP // BLK

F32 = jnp.float32


def _tc1_body(x_ref, w_ref, a_ref, wh_ref, s_ref):
    wh = jnp.dot(x_ref[...], w_ref[...], preferred_element_type=F32)
    wh_ref[...] = wh
    a = a_ref[...]
    sa = jnp.dot(wh, a[:H1, :], preferred_element_type=F32)
    sb = jnp.dot(wh, a[H1:, :], preferred_element_type=F32)
    s_ref[...] = jnp.concatenate([sa, sb], axis=1)


def _tc1(x, W1, a1):
    return pl.pallas_call(
        _tc1_body,
        grid=(GRID,),
        in_specs=[
            pl.BlockSpec((BLK, D_IN), lambda i: (i, 0)),
            pl.BlockSpec((D_IN, H1), lambda i: (0, 0)),
            pl.BlockSpec((2 * H1, 1), lambda i: (0, 0)),
        ],
        out_specs=[
            pl.BlockSpec((BLK, H1), lambda i: (i, 0)),
            pl.BlockSpec((BLK, 2), lambda i: (i, 0)),
        ],
        out_shape=[
            jax.ShapeDtypeStruct((NP, H1), F32),
            jax.ShapeDtypeStruct((NP, 2), F32),
        ],
    )(x, W1, a1)


def _tc2_body(hp_ref, w2_ref, w3_ref, a2_ref, a3_ref, wh_ref, s_ref):
    h = jnp.maximum(hp_ref[0] + hp_ref[1], 0.0)
    w23 = jnp.concatenate([w2_ref[...], w3_ref[...]], axis=1)
    wh = jnp.dot(h, w23, preferred_element_type=F32)
    wh_ref[...] = wh
    a2 = a2_ref[...]
    a3 = a3_ref[...]
    s2a = jnp.dot(wh[:, :H2], a2[:H2, :], preferred_element_type=F32)
    s2b = jnp.dot(wh[:, :H2], a2[H2:, :], preferred_element_type=F32)
    s3a = jnp.dot(wh[:, H2:], a3[:H2, :], preferred_element_type=F32)
    s3b = jnp.dot(wh[:, H2:], a3[H2:, :], preferred_element_type=F32)
    s_ref[...] = jnp.concatenate([s2a, s2b, s3a, s3b], axis=1)


def _tc2(hp1, W2, W3, a2, a3):
    return pl.pallas_call(
        _tc2_body,
        grid=(GRID,),
        in_specs=[
            pl.BlockSpec((NC, BLK, H1), lambda i: (0, i, 0)),
            pl.BlockSpec((H1, H2), lambda i: (0, 0)),
            pl.BlockSpec((H1, H2), lambda i: (0, 0)),
            pl.BlockSpec((2 * H2, 1), lambda i: (0, 0)),
            pl.BlockSpec((2 * H2, 1), lambda i: (0, 0)),
        ],
        out_specs=[
            pl.BlockSpec((BLK, 2 * H2), lambda i: (i, 0)),
            pl.BlockSpec((BLK, 4), lambda i: (i, 0)),
        ],
        out_shape=[
            jax.ShapeDtypeStruct((NP, 2 * H2), F32),
            jax.ShapeDtypeStruct((NP, 4), F32),
        ],
    )(hp1, W2, W3, a2, a3)


def _tc3_body(hp_ref, mu_ref, lv_ref):
    m = hp_ref[0] + hp_ref[1]
    mu_ref[...] = m[:, :H2]
    lv_ref[...] = m[:, H2:]


def _tc3(hp23):
    return pl.pallas_call(
        _tc3_body,
        grid=(GRID,),
        in_specs=[pl.BlockSpec((NC, BLK, 2 * H2), lambda i: (0, i, 0))],
        out_specs=[
            pl.BlockSpec((BLK, H2), lambda i: (i, 0)),
            pl.BlockSpec((BLK, H2), lambda i: (i, 0)),
        ],
        out_shape=[
            jax.ShapeDtypeStruct((NP, H2), F32),
            jax.ShapeDtypeStruct((NP, H2), F32),
        ],
    )(hp23)


def _make_sc(nch):
    hw = 64
    st = 2 * nch * N
    oth_blk = 20
    mesh = plsc.VectorSubcoreMesh(core_axis_name="c", subcore_axis_name="s")

    scratch = [
        pltpu.VMEM((st,), F32),
        pltpu.VMEM((RPT + oth_blk, 128), jnp.int32),
        pltpu.VMEM((RPT + oth_blk, 128), jnp.int32),
        pltpu.VMEM((nch, 128), F32),
        pltpu.VMEM((nch, 128), F32),
        pltpu.VMEM((1, 128), jnp.int32),
        pltpu.VMEM((1, 128), jnp.int32),
        pltpu.VMEM((nch * NPS,), F32),
        pltpu.VMEM((128, hw), F32),
        pltpu.VMEM((128, hw), F32),
        pltpu.VMEM_SHARED((nch * NP,), F32),
        pltpu.VMEM_SHARED((NP, hw), F32),
        pltpu.SemaphoreType.DMA,
        pltpu.SemaphoreType.DMA,
        pltpu.SemaphoreType.DMA,
        pltpu.SemaphoreType.DMA,
        pltpu.SemaphoreType.DMA,
    ]

    @functools.partial(
        pl.kernel,
        out_type=jax.ShapeDtypeStruct((NC, NP, hw), F32),
        mesh=mesh,
        scratch_types=scratch,
        compiler_params=pltpu.CompilerParams(use_tc_tiling_on_sc=False,
                                             needs_layout_passes=False),
    )
    def sc_kernel(src_hbm, dst_hbm, s_hbm, wh_hbm, hp_out,
                  s_v, src_v, dst_v, exrA_v, exrB_v, idx1A_v, idx1B_v,
                  den_v, rowsA_v, rowsB_v, den_sh, hp_sh,
                  gsA, gsB, ssA, ssB, sem):
        cid = lax.axis_index("c")
        sid = lax.axis_index("s")

        own = (cid * NS + sid) * RPT
        oth = ((1 - cid) * NS + sid) * RPT
        pltpu.sync_copy(s_hbm, s_v)
        pltpu.sync_copy(src_hbm.at[pl.ds(own, RPT)], src_v.at[pl.ds(0, RPT)])
        pltpu.sync_copy(dst_hbm.at[pl.ds(own, RPT)], dst_v.at[pl.ds(0, RPT)])

        def zrow(j, c):
            for q in range(hw // 16):
                rowsA_v[j, pl.ds(q * 16, 16)] = jnp.zeros((16,), F32)
            return c
        lax.fori_loop(0, 128, zrow, 0)

        def zden(k, c):
            den_v[pl.ds(k * 16, 16)] = jnp.zeros((16,), F32)
            return c
        lax.fori_loop(0, nch * NPS // 16, zden, 0)
        for ch in range(nch):
            pltpu.sync_copy(den_v.at[pl.ds(ch * NPS, NPS)],
                            den_sh.at[pl.ds(ch * NP + sid * NPS, NPS)])
        for o in range(0, NPS, 128):
            pltpu.sync_copy(rowsA_v, hp_sh.at[pl.ds(sid * NPS + o, 128)])
        plsc.subcore_barrier()

        def compute_ex(r, exr_v, idx1_v):
            for k in range(8):
                sl = pl.ds(k * 16, 16)
                s16 = src_v[r, sl]
                d16 = dst_v[r, sl]
                if nch == 1:
                    g0 = plsc.load_gather(s_v, [s16 * 2])
                    g1 = plsc.load_gather(s_v, [d16 * 2 + 1])
                    l0 = g0 + g1
                    l0 = jnp.where(l0 >= 0.0, l0, l0 * 0.01) * 2.0
                    exr_v[0, sl] = jnp.exp(jnp.minimum(l0, 80.0))
                else:
                    g0 = plsc.load_gather(s_v, [s16 * 4])
                    g1 = plsc.load_gather(s_v, [d16 * 4 + 1])
                    g2 = plsc.load_gather(s_v, [s16 * 4 + 2])
                    g3 = plsc.load_gather(s_v, [d16 * 4 + 3])
                    l0 = g0 + g1
                    l0 = jnp.where(l0 >= 0.0, l0, l0 * 0.01) * 2.0
                    l1 = g2 + g3
                    l1 = jnp.where(l1 >= 0.0, l1, l1 * 0.01) * 2.0
                    exr_v[0, sl] = jnp.exp(jnp.minimum(l0, 80.0))
                    exr_v[1, sl] = jnp.exp(jnp.minimum(l1, 80.0))
                    idx1_v[0, sl] = s16 + NP

        def den_scatter(r, exr_v, idx1_v):
            pltpu.sync_copy(exr_v.at[0], den_sh.at[src_v.at[r]], add=True)
            if nch == 2:
                pltpu.sync_copy(exr_v.at[1], den_sh.at[idx1_v.at[0]], add=True)

        def scale(rows_v, exr_v):
            for k in range(8):
                sl = pl.ds(k * 16, 16)
                e0 = exr_v[0, sl]
                e1 = exr_v[1, sl] if nch == 2 else e0
                for jj in range(16):
                    j = k * 16 + jj
                    a0 = e0[jj]
                    a1 = e1[jj]
                    for q in range(2):
                        qs = pl.ds(q * 16, 16)
                        rows_v[j, qs] = rows_v[j, qs] * a0
                    for q in range(2, 4):
                        qs = pl.ds(q * 16, 16)
                        rows_v[j, qs] = rows_v[j, qs] * a1

        n_own = jnp.clip(R_E - own, 0, RPT)
        n_oth = jnp.clip(R_E - oth, 0, RPT)

        pltpu.async_copy(wh_hbm.at[dst_v.at[0]], rowsA_v, gsA)
        pltpu.async_copy(wh_hbm.at[dst_v.at[1]], rowsB_v, gsB)
        compute_ex(0, exrA_v, idx1A_v)
        den_scatter(0, exrA_v, idx1A_v)
        compute_ex(1, exrB_v, idx1B_v)
        den_scatter(1, exrB_v, idx1B_v)

        n2 = n_own // 2

        def own_body(r2, c):
            a = 2 * r2
            b = a + 1
            pltpu.make_async_copy(wh_hbm.at[dst_v.at[a]], rowsA_v, gsA).wait()
            scale(rowsA_v, exrA_v)
            pltpu.async_copy(rowsA_v, hp_sh.at[src_v.at[a]], ssA, add=True)
            pltpu.make_async_copy(wh_hbm.at[dst_v.at[b]], rowsB_v, gsB).wait()
            scale(rowsB_v, exrB_v)
            pltpu.async_copy(rowsB_v, hp_sh.at[src_v.at[b]], ssB, add=True)

            @pl.when(r2 + 1 < n2)
            def _prefetch():
                a2 = a + 2
                b2 = b + 2
                pltpu.make_async_copy(rowsA_v, hp_sh.at[src_v.at[a]],
                                      ssA).wait()
                pltpu.async_copy(wh_hbm.at[dst_v.at[a2]], rowsA_v, gsA)
                compute_ex(a2, exrA_v, idx1A_v)
                den_scatter(a2, exrA_v, idx1A_v)
                pltpu.make_async_copy(rowsB_v, hp_sh.at[src_v.at[b]],
                                      ssB).wait()
                pltpu.async_copy(wh_hbm.at[dst_v.at[b2]], rowsB_v, gsB)
                compute_ex(b2, exrB_v, idx1B_v)
                den_scatter(b2, exrB_v, idx1B_v)
            return c

        lax.fori_loop(0, n2, own_body, 0)
        pltpu.make_async_copy(rowsA_v, hp_sh.at[pl.ds(0, 128)], ssA).wait()
        pltpu.make_async_copy(rowsB_v, hp_sh.at[pl.ds(0, 128)], ssB).wait()

        def oth_body(r, c):
            compute_ex(RPT + r, exrA_v, idx1A_v)
            den_scatter(RPT + r, exrA_v, idx1A_v)
            return c

        for half in range(RPT // oth_blk):
            pltpu.sync_copy(src_hbm.at[pl.ds(oth + half * oth_blk, oth_blk)],
                            src_v.at[pl.ds(RPT, oth_blk)])
            pltpu.sync_copy(dst_hbm.at[pl.ds(oth + half * oth_blk, oth_blk)],
                            dst_v.at[pl.ds(RPT, oth_blk)])
            cnt = jnp.clip(n_oth - half * oth_blk, 0, oth_blk)
            lax.fori_loop(0, cnt, oth_body, 0)
        plsc.subcore_barrier()

        for ch in range(nch):
            pltpu.sync_copy(den_sh.at[pl.ds(ch * NP + sid * NPS, NPS)],
                            den_v.at[pl.ds(ch * NPS, NPS)])
        base = sid * NPS

        def norm_body(b, c):
            rows_v = rowsA_v
            pltpu.async_copy(hp_sh.at[pl.ds(base + b * 128, 128)], rows_v,
                             sem).wait()
            for k in range(8):
                sl = pl.ds(b * 128 + k * 16, 16)
                d0 = den_v[sl]
                r0 = 1.0 / jnp.maximum(d0, 1e-30)
                if nch == 2:
                    d1 = den_v[pl.ds(NPS + b * 128 + k * 16, 16)]
                    r1 = 1.0 / jnp.maximum(d1, 1e-30)
                else:
                    r1 = r0
                for jj in range(16):
                    j = k * 16 + jj
                    a0 = r0[jj]
                    a1 = r1[jj]
                    for q in range(2):
                        qs = pl.ds(q * 16, 16)
                        rows_v[j, qs] = rows_v[j, qs] * a0
                    for q in range(2, 4):
                        qs = pl.ds(q * 16, 16)
                        rows_v[j, qs] = rows_v[j, qs] * a1
            pltpu.sync_copy(rows_v,
                            hp_out.at[cid, pl.ds(base + b * 128, 128)])
            return c

        lax.fori_loop(0, NPS // 128, norm_body, 0)

    return sc_kernel


_sc1 = _make_sc(1)
_sc23 = _make_sc(2)


def kernel(x, edge_index, W1, a1, W2, a2, W3, a3):
    src2d = jnp.pad(edge_index[0].reshape(R_E, 128), ((0, R_P - R_E), (0, 0)))
    dst2d = jnp.pad(edge_index[1].reshape(R_E, 128), ((0, R_P - R_E), (0, 0)))
    x_p = jnp.pad(x, ((0, NP - N), (0, 0)))

    wh1, s1 = _tc1(x_p, W1, a1)
    hp1 = _sc1(src2d, dst2d, s1.reshape(-1)[:2 * N], wh1)
    wh23, s23 = _tc2(hp1, W2, W3, a2, a3)
    hp23 = _sc23(src2d, dst2d, s23.reshape(-1)[:4 * N], wh23)
    mu, logvar = _tc3(hp23)
    return (mu[:N], mu[:N], logvar[:N])


# speedup vs baseline: 44.2690x; 1.0500x over previous
import functools

import jax
import jax.numpy as jnp
from jax import lax
from jax.experimental import pallas as pl
from jax.experimental.pallas import tpu as pltpu
from jax.experimental.pallas import tpu_sc as plsc

N = 10000
E = 320000
D_IN, H1, H2 = 128, 64, 32

NC, NS = 2, 16
R_E = E // 128
RPT = 80
R_P = NC * NS * RPT
NP = 10240
NPS = NP // NS
BLK = 1024
GRID = NP // BLK

F32 = jnp.float32


def _tc1_body(x_ref, w_ref, a_ref, wh_ref, s_ref):
    wh = jnp.dot(x_ref[...], w_ref[...], preferred_element_type=F32)
    wh_ref[...] = wh
    a = a_ref[...]
    sa = jnp.dot(wh, a[:H1, :], preferred_element_type=F32)
    sb = jnp.dot(wh, a[H1:, :], preferred_element_type=F32)
    s_ref[...] = jnp.concatenate([sa, sb], axis=1)


def _tc1(x, W1, a1):
    return pl.pallas_call(
        _tc1_body,
        grid=(GRID,),
        in_specs=[
            pl.BlockSpec((BLK, D_IN), lambda i: (i, 0)),
            pl.BlockSpec((D_IN, H1), lambda i: (0, 0)),
            pl.BlockSpec((2 * H1, 1), lambda i: (0, 0)),
        ],
        out_specs=[
            pl.BlockSpec((BLK, H1), lambda i: (i, 0)),
            pl.BlockSpec((BLK, 2), lambda i: (i, 0)),
        ],
        out_shape=[
            jax.ShapeDtypeStruct((NP, H1), F32),
            jax.ShapeDtypeStruct((NP, 2), F32),
        ],
    )(x, W1, a1)


def _tc2_body(hp_ref, w2_ref, w3_ref, a2_ref, a3_ref, wh_ref, s_ref):
    h = jnp.maximum(hp_ref[0] + hp_ref[1], 0.0)
    w23 = jnp.concatenate([w2_ref[...], w3_ref[...]], axis=1)
    wh = jnp.dot(h, w23, preferred_element_type=F32)
    wh_ref[...] = wh
    a2 = a2_ref[...]
    a3 = a3_ref[...]
    s2a = jnp.dot(wh[:, :H2], a2[:H2, :], preferred_element_type=F32)
    s2b = jnp.dot(wh[:, :H2], a2[H2:, :], preferred_element_type=F32)
    s3a = jnp.dot(wh[:, H2:], a3[:H2, :], preferred_element_type=F32)
    s3b = jnp.dot(wh[:, H2:], a3[H2:, :], preferred_element_type=F32)
    s_ref[...] = jnp.concatenate([s2a, s2b, s3a, s3b], axis=1)


def _tc2(hp1, W2, W3, a2, a3):
    return pl.pallas_call(
        _tc2_body,
        grid=(GRID,),
        in_specs=[
            pl.BlockSpec((NC, BLK, H1), lambda i: (0, i, 0)),
            pl.BlockSpec((H1, H2), lambda i: (0, 0)),
            pl.BlockSpec((H1, H2), lambda i: (0, 0)),
            pl.BlockSpec((2 * H2, 1), lambda i: (0, 0)),
            pl.BlockSpec((2 * H2, 1), lambda i: (0, 0)),
        ],
        out_specs=[
            pl.BlockSpec((BLK, 2 * H2), lambda i: (i, 0)),
            pl.BlockSpec((BLK, 4), lambda i: (i, 0)),
        ],
        out_shape=[
            jax.ShapeDtypeStruct((NP, 2 * H2), F32),
            jax.ShapeDtypeStruct((NP, 4), F32),
        ],
    )(hp1, W2, W3, a2, a3)


def _tc3_body(hp_ref, mu_ref, lv_ref):
    m = hp_ref[0] + hp_ref[1]
    mu_ref[...] = m[:, :H2]
    lv_ref[...] = m[:, H2:]


def _tc3(hp23):
    return pl.pallas_call(
        _tc3_body,
        grid=(GRID,),
        in_specs=[pl.BlockSpec((NC, BLK, 2 * H2), lambda i: (0, i, 0))],
        out_specs=[
            pl.BlockSpec((BLK, H2), lambda i: (i, 0)),
            pl.BlockSpec((BLK, H2), lambda i: (i, 0)),
        ],
        out_shape=[
            jax.ShapeDtypeStruct((NP, H2), F32),
            jax.ShapeDtypeStruct((NP, H2), F32),
        ],
    )(hp23)


def _make_sc(nch):
    hw = 64
    st = 2 * nch * N
    oth_blk = 20
    mesh = plsc.VectorSubcoreMesh(core_axis_name="c", subcore_axis_name="s")

    scratch = [
        pltpu.VMEM((st,), F32),
        pltpu.VMEM((RPT + oth_blk, 128), jnp.int32),
        pltpu.VMEM((RPT + oth_blk, 128), jnp.int32),
        pltpu.VMEM((nch, 128), F32),
        pltpu.VMEM((nch, 128), F32),
        pltpu.VMEM((1, 128), jnp.int32),
        pltpu.VMEM((1, 128), jnp.int32),
        pltpu.VMEM((nch * NPS,), F32),
        pltpu.VMEM((128, hw), F32),
        pltpu.VMEM((128, hw), F32),
        pltpu.VMEM_SHARED((nch * NP,), F32),
        pltpu.VMEM_SHARED((NP, hw), F32),
        pltpu.SemaphoreType.DMA,
        pltpu.SemaphoreType.DMA,
        pltpu.SemaphoreType.DMA,
        pltpu.SemaphoreType.DMA,
        pltpu.SemaphoreType.DMA,
        pltpu.SemaphoreType.DMA,
        pltpu.SemaphoreType.DMA,
    ]

    @functools.partial(
        pl.kernel,
        out_type=jax.ShapeDtypeStruct((NC, NP, hw), F32),
        mesh=mesh,
        scratch_types=scratch,
        compiler_params=pltpu.CompilerParams(use_tc_tiling_on_sc=False,
                                             needs_layout_passes=False),
    )
    def sc_kernel(src_hbm, dst_hbm, s_hbm, wh_hbm, hp_out,
                  s_v, src_v, dst_v, exrA_v, exrB_v, idx1A_v, idx1B_v,
                  den_v, rowsA_v, rowsB_v, den_sh, hp_sh,
                  gsA, gsB, ssA, ssB, dsA, dsB, sem):
        cid = lax.axis_index("c")
        sid = lax.axis_index("s")

        own = (cid * NS + sid) * RPT
        oth = ((1 - cid) * NS + sid) * RPT
        pltpu.sync_copy(s_hbm, s_v)
        pltpu.sync_copy(src_hbm.at[pl.ds(own, RPT)], src_v.at[pl.ds(0, RPT)])
        pltpu.sync_copy(dst_hbm.at[pl.ds(own, RPT)], dst_v.at[pl.ds(0, RPT)])

        def zrow(j, c):
            for q in range(hw // 16):
                rowsA_v[j, pl.ds(q * 16, 16)] = jnp.zeros((16,), F32)
            return c
        lax.fori_loop(0, 128, zrow, 0)

        def zden(k, c):
            den_v[pl.ds(k * 16, 16)] = jnp.zeros((16,), F32)
            return c
        lax.fori_loop(0, nch * NPS // 16, zden, 0)
        for ch in range(nch):
            pltpu.sync_copy(den_v.at[pl.ds(ch * NPS, NPS)],
                            den_sh.at[pl.ds(ch * NP + sid * NPS, NPS)])
        for o in range(0, NPS, 128):
            pltpu.sync_copy(rowsA_v, hp_sh.at[pl.ds(sid * NPS + o, 128)])
        plsc.subcore_barrier()

        def compute_ex(r, exr_v, idx1_v):
            for k in range(8):
                sl = pl.ds(k * 16, 16)
                s16 = src_v[r, sl]
                d16 = dst_v[r, sl]
                if nch == 1:
                    g0 = plsc.load_gather(s_v, [s16 * 2])
                    g1 = plsc.load_gather(s_v, [d16 * 2 + 1])
                    l0 = g0 + g1
                    l0 = jnp.where(l0 >= 0.0, l0, l0 * 0.01) * 2.0
                    exr_v[0, sl] = jnp.exp(jnp.minimum(l0, 80.0))
                else:
                    g0 = plsc.load_gather(s_v, [s16 * 4])
                    g1 = plsc.load_gather(s_v, [d16 * 4 + 1])
                    g2 = plsc.load_gather(s_v, [s16 * 4 + 2])
                    g3 = plsc.load_gather(s_v, [d16 * 4 + 3])
                    l0 = g0 + g1
                    l0 = jnp.where(l0 >= 0.0, l0, l0 * 0.01) * 2.0
                    l1 = g2 + g3
                    l1 = jnp.where(l1 >= 0.0, l1, l1 * 0.01) * 2.0
                    exr_v[0, sl] = jnp.exp(jnp.minimum(l0, 80.0))
                    exr_v[1, sl] = jnp.exp(jnp.minimum(l1, 80.0))
                    idx1_v[0, sl] = s16 + NP

        def den_scatter(r, exr_v, idx1_v, dsem):
            pltpu.async_copy(exr_v.at[0], den_sh.at[src_v.at[r]], dsem,
                             add=True)
            if nch == 2:
                pltpu.async_copy(exr_v.at[1], den_sh.at[idx1_v.at[0]], dsem,
                                 add=True)

        def den_drain(dsem):
            for _ in range(nch):
                pltpu.make_async_copy(exrA_v.at[0],
                                      den_sh.at[pl.ds(0, 128)], dsem).wait()

        def scale(rows_v, exr_v):
            for k in range(8):
                sl = pl.ds(k * 16, 16)
                e0 = exr_v[0, sl]
                e1 = exr_v[1, sl] if nch == 2 else e0
                for jj in range(16):
                    j = k * 16 + jj
                    a0 = e0[jj]
                    a1 = e1[jj]
                    for q in range(2):
                        qs = pl.ds(q * 16, 16)
                        rows_v[j, qs] = rows_v[j, qs] * a0
                    for q in range(2, 4):
                        qs = pl.ds(q * 16, 16)
                        rows_v[j, qs] = rows_v[j, qs] * a1

        n_own = jnp.clip(R_E - own, 0, RPT)
        n_oth = jnp.clip(R_E - oth, 0, RPT)

        pltpu.async_copy(wh_hbm.at[dst_v.at[0]], rowsA_v, gsA)
        pltpu.async_copy(wh_hbm.at[dst_v.at[1]], rowsB_v, gsB)
        compute_ex(0, exrA_v, idx1A_v)
        den_scatter(0, exrA_v, idx1A_v, dsA)
        compute_ex(1, exrB_v, idx1B_v)
        den_scatter(1, exrB_v, idx1B_v, dsB)

        n2 = n_own // 2

        def own_body(r2, c):
            a = 2 * r2
            b = a + 1
            pltpu.make_async_copy(wh_hbm.at[dst_v.at[a]], rowsA_v, gsA).wait()
            scale(rowsA_v, exrA_v)
            pltpu.async_copy(rowsA_v, hp_sh.at[src_v.at[a]], ssA, add=True)
            pltpu.make_async_copy(wh_hbm.at[dst_v.at[b]], rowsB_v, gsB).wait()
            scale(rowsB_v, exrB_v)
            pltpu.async_copy(rowsB_v, hp_sh.at[src_v.at[b]], ssB, add=True)

            @pl.when(r2 + 1 < n2)
            def _prefetch():
                a2 = a + 2
                b2 = b + 2
                pltpu.make_async_copy(rowsA_v, hp_sh.at[src_v.at[a]],
                                      ssA).wait()
                pltpu.async_copy(wh_hbm.at[dst_v.at[a2]], rowsA_v, gsA)
                den_drain(dsA)
                compute_ex(a2, exrA_v, idx1A_v)
                den_scatter(a2, exrA_v, idx1A_v, dsA)
                pltpu.make_async_copy(rowsB_v, hp_sh.at[src_v.at[b]],
                                      ssB).wait()
                pltpu.async_copy(wh_hbm.at[dst_v.at[b2]], rowsB_v, gsB)
                den_drain(dsB)
                compute_ex(b2, exrB_v, idx1B_v)
                den_scatter(b2, exrB_v, idx1B_v, dsB)
            return c

        lax.fori_loop(0, n2, own_body, 0)
        pltpu.make_async_copy(rowsA_v, hp_sh.at[pl.ds(0, 128)], ssA).wait()
        pltpu.make_async_copy(rowsB_v, hp_sh.at[pl.ds(0, 128)], ssB).wait()

        def oth_body(i, c):
            r = RPT + 2 * i
            den_drain(dsA)
            compute_ex(r, exrA_v, idx1A_v)
            den_scatter(r, exrA_v, idx1A_v, dsA)
            den_drain(dsB)
            compute_ex(r + 1, exrB_v, idx1B_v)
            den_scatter(r + 1, exrB_v, idx1B_v, dsB)
            return c

        for half in range(RPT // oth_blk):
            pltpu.sync_copy(src_hbm.at[pl.ds(oth + half * oth_blk, oth_blk)],
                            src_v.at[pl.ds(RPT, oth_blk)])
            pltpu.sync_copy(dst_hbm.at[pl.ds(oth + half * oth_blk, oth_blk)],
                            dst_v.at[pl.ds(RPT, oth_blk)])
            cnt = jnp.clip(n_oth - half * oth_blk, 0, oth_blk)
            lax.fori_loop(0, cnt // 2, oth_body, 0)
        den_drain(dsA)
        den_drain(dsB)
        plsc.subcore_barrier()

        for ch in range(nch):
            pltpu.sync_copy(den_sh.at[pl.ds(ch * NP + sid * NPS, NPS)],
                            den_v.at[pl.ds(ch * NPS, NPS)])
        base = sid * NPS

        def norm_body(b, c):
            rows_v = rowsA_v
            pltpu.async_copy(hp_sh.at[pl.ds(base + b * 128, 128)], rows_v,
                             sem).wait()
            for k in range(8):
                sl = pl.ds(b * 128 + k * 16, 16)
                d0 = den_v[sl]
                r0 = 1.0 / jnp.maximum(d0, 1e-30)
                if nch == 2:
                    d1 = den_v[pl.ds(NPS + b * 128 + k * 16, 16)]
                    r1 = 1.0 / jnp.maximum(d1, 1e-30)
                else:
                    r1 = r0
                for jj in range(16):
                    j = k * 16 + jj
                    a0 = r0[jj]
                    a1 = r1[jj]
                    for q in range(2):
                        qs = pl.ds(q * 16, 16)
                        rows_v[j, qs] = rows_v[j, qs] * a0
                    for q in range(2, 4):
                        qs = pl.ds(q * 16, 16)
                        rows_v[j, qs] = rows_v[j, qs] * a1
            pltpu.sync_copy(rows_v,
                            hp_out.at[cid, pl.ds(base + b * 128, 128)])
            return c

        lax.fori_loop(0, NPS // 128, norm_body, 0)

    return sc_kernel


_sc1 = _make_sc(1)
_sc23 = _make_sc(2)


def kernel(x, edge_index, W1, a1, W2, a2, W3, a3):
    src2d = jnp.pad(edge_index[0].reshape(R_E, 128), ((0, R_P - R_E), (0, 0)))
    dst2d = jnp.pad(edge_index[1].reshape(R_E, 128), ((0, R_P - R_E), (0, 0)))
    x_p = jnp.pad(x, ((0, NP - N), (0, 0)))

    wh1, s1 = _tc1(x_p, W1, a1)
    hp1 = _sc1(src2d, dst2d, s1.reshape(-1)[:2 * N], wh1)
    wh23, s23 = _tc2(hp1, W2, W3, a2, a3)
    hp23 = _sc23(src2d, dst2d, s23.reshape(-1)[:4 * N], wh23)
    mu, logvar = _tc3(hp23)
    return (mu[:N], mu[:N], logvar[:N])


# speedup vs baseline: 49.7336x; 1.1234x over previous
import functools

import jax
import jax.numpy as jnp
from jax import lax
from jax.experimental import pallas as pl
from jax.experimental.pallas import tpu as pltpu
from jax.experimental.pallas import tpu_sc as plsc

N = 10000
E = 320000
D_IN, H1, H2 = 128, 64, 32

NC, NS = 2, 16
R_E = E // 128
RPT = 160
R_P = NS * RPT
NP = 10240
NPS = NP // NS
HW = 32
BLK = 1024
GRID = NP // BLK

F32 = jnp.float32


def _tc1_body(x_ref, w_ref, a_ref, wh_ref, s_ref):
    wh = jnp.dot(x_ref[...], w_ref[...], preferred_element_type=F32)
    wh_ref[0] = wh[:, :HW]
    wh_ref[1] = wh[:, HW:]
    a = a_ref[...]
    sa = jnp.dot(wh, a[:H1, :], preferred_element_type=F32)
    sb = jnp.dot(wh, a[H1:, :], preferred_element_type=F32)
    s_ref[...] = jnp.concatenate([sa, sb], axis=1)


def _tc1(x, W1, a1):
    return pl.pallas_call(
        _tc1_body,
        grid=(GRID,),
        in_specs=[
            pl.BlockSpec((BLK, D_IN), lambda i: (i, 0)),
            pl.BlockSpec((D_IN, H1), lambda i: (0, 0)),
            pl.BlockSpec((2 * H1, 1), lambda i: (0, 0)),
        ],
        out_specs=[
            pl.BlockSpec((NC, BLK, HW), lambda i: (0, i, 0)),
            pl.BlockSpec((BLK, 2), lambda i: (i, 0)),
        ],
        out_shape=[
            jax.ShapeDtypeStruct((NC, NP, HW), F32),
            jax.ShapeDtypeStruct((NP, 2), F32),
        ],
    )(x, W1, a1)


def _tc2_body(hp_ref, w2_ref, w3_ref, a2_ref, a3_ref, wh_ref, s_ref):
    h = jnp.maximum(jnp.concatenate([hp_ref[0], hp_ref[1]], axis=1), 0.0)
    w23 = jnp.concatenate([w2_ref[...], w3_ref[...]], axis=1)
    wh = jnp.dot(h, w23, preferred_element_type=F32)
    wh_ref[0] = wh[:, :H2]
    wh_ref[1] = wh[:, H2:]
    a2 = a2_ref[...]
    a3 = a3_ref[...]
    s2a = jnp.dot(wh[:, :H2], a2[:H2, :], preferred_element_type=F32)
    s2b = jnp.dot(wh[:, :H2], a2[H2:, :], preferred_element_type=F32)
    s3a = jnp.dot(wh[:, H2:], a3[:H2, :], preferred_element_type=F32)
    s3b = jnp.dot(wh[:, H2:], a3[H2:, :], preferred_element_type=F32)
    s_ref[...] = jnp.concatenate([s2a, s2b, s3a, s3b], axis=1)


def _tc2(hp1, W2, W3, a2, a3):
    return pl.pallas_call(
        _tc2_body,
        grid=(GRID,),
        in_specs=[
            pl.BlockSpec((NC, BLK, HW), lambda i: (0, i, 0)),
            pl.BlockSpec((H1, H2), lambda i: (0, 0)),
            pl.BlockSpec((H1, H2), lambda i: (0, 0)),
            pl.BlockSpec((2 * H2, 1), lambda i: (0, 0)),
            pl.BlockSpec((2 * H2, 1), lambda i: (0, 0)),
        ],
        out_specs=[
            pl.BlockSpec((NC, BLK, HW), lambda i: (0, i, 0)),
            pl.BlockSpec((BLK, 4), lambda i: (i, 0)),
        ],
        out_shape=[
            jax.ShapeDtypeStruct((NC, NP, HW), F32),
            jax.ShapeDtypeStruct((NP, 4), F32),
        ],
    )(hp1, W2, W3, a2, a3)


def _make_sc(stride):
    st = stride * N
    mesh = plsc.VectorSubcoreMesh(core_axis_name="c", subcore_axis_name="s")

    scratch = [
        pltpu.VMEM((st,), F32),
        pltpu.VMEM((RPT, 128), jnp.int32),
        pltpu.VMEM((RPT, 128), jnp.int32),
        pltpu.VMEM((1, 128), F32),
        pltpu.VMEM((1, 128), F32),
        pltpu.VMEM((NPS,), F32),
        pltpu.VMEM((128, HW), F32),
        pltpu.VMEM((128, HW), F32),
        pltpu.VMEM_SHARED((NP,), F32),
        pltpu.VMEM_SHARED((NP, HW), F32),
        pltpu.SemaphoreType.DMA,
        pltpu.SemaphoreType.DMA,
        pltpu.SemaphoreType.DMA,
        pltpu.SemaphoreType.DMA,
        pltpu.SemaphoreType.DMA,
        pltpu.SemaphoreType.DMA,
        pltpu.SemaphoreType.DMA,
    ]

    @functools.partial(
        pl.kernel,
        out_type=jax.ShapeDtypeStruct((NC, NP, HW), F32),
        mesh=mesh,
        scratch_types=scratch,
        compiler_params=pltpu.CompilerParams(use_tc_tiling_on_sc=False,
                                             needs_layout_passes=False),
    )
    def sc_kernel(src_hbm, dst_hbm, s_hbm, wh_hbm, hp_out,
                  s_v, src_v, dst_v, exrA_v, exrB_v,
                  den_v, rowsA_v, rowsB_v, den_sh, hp_sh,
                  gsA, gsB, ssA, ssB, dsA, dsB, sem):
        cid = lax.axis_index("c")
        sid = lax.axis_index("s")
        off = cid * (stride - 2)

        pltpu.sync_copy(s_hbm, s_v)
        pltpu.sync_copy(src_hbm.at[pl.ds(sid * RPT, RPT)], src_v)
        pltpu.sync_copy(dst_hbm.at[pl.ds(sid * RPT, RPT)], dst_v)

        def zrow(j, c):
            for q in range(HW // 16):
                rowsA_v[j, pl.ds(q * 16, 16)] = jnp.zeros((16,), F32)
            return c
        lax.fori_loop(0, 128, zrow, 0)

        def zden(k, c):
            den_v[pl.ds(k * 16, 16)] = jnp.zeros((16,), F32)
            return c
        lax.fori_loop(0, NPS // 16, zden, 0)
        pltpu.sync_copy(den_v, den_sh.at[pl.ds(sid * NPS, NPS)])
        for o in range(0, NPS, 128):
            pltpu.sync_copy(rowsA_v, hp_sh.at[pl.ds(sid * NPS + o, 128)])
        plsc.subcore_barrier()

        def compute_ex(r, exr_v):
            for k in range(8):
                sl = pl.ds(k * 16, 16)
                s16 = src_v[r, sl]
                d16 = dst_v[r, sl]
                g0 = plsc.load_gather(s_v, [s16 * stride + off])
                g1 = plsc.load_gather(s_v, [d16 * stride + (off + 1)])
                l0 = g0 + g1
                l0 = jnp.where(l0 >= 0.0, l0, l0 * 0.01) * 2.0
                exr_v[0, sl] = jnp.exp(jnp.minimum(l0, 80.0))

        def den_scatter(r, exr_v, dsem):
            pltpu.async_copy(exr_v.at[0], den_sh.at[src_v.at[r]], dsem,
                             add=True)

        def den_drain(dsem):
            pltpu.make_async_copy(exrA_v.at[0],
                                  den_sh.at[pl.ds(0, 128)], dsem).wait()

        def scale(rows_v, exr_v):
            for k in range(8):
                sl = pl.ds(k * 16, 16)
                e0 = exr_v[0, sl]
                for jj in range(16):
                    j = k * 16 + jj
                    a0 = e0[jj]
                    for q in range(HW // 16):
                        qs = pl.ds(q * 16, 16)
                        rows_v[j, qs] = rows_v[j, qs] * a0

        n_rows = jnp.clip(R_E - sid * RPT, 0, RPT)

        pltpu.async_copy(wh_hbm.at[cid].at[dst_v.at[0]], rowsA_v, gsA)
        pltpu.async_copy(wh_hbm.at[cid].at[dst_v.at[1]], rowsB_v, gsB)
        compute_ex(0, exrA_v)
        den_scatter(0, exrA_v, dsA)
        compute_ex(1, exrB_v)
        den_scatter(1, exrB_v, dsB)

        n2 = n_rows // 2

        def body(r2, c):
            a = 2 * r2
            b = a + 1
            pltpu.make_async_copy(wh_hbm.at[cid].at[dst_v.at[a]], rowsA_v,
                                  gsA).wait()
            scale(rowsA_v, exrA_v)
            pltpu.async_copy(rowsA_v, hp_sh.at[src_v.at[a]], ssA, add=True)
            pltpu.make_async_copy(wh_hbm.at[cid].at[dst_v.at[b]], rowsB_v,
                                  gsB).wait()
            scale(rowsB_v, exrB_v)
            pltpu.async_copy(rowsB_v, hp_sh.at[src_v.at[b]], ssB, add=True)

            @pl.when(r2 + 1 < n2)
            def _prefetch():
                a2 = a + 2
                b2 = b + 2
                pltpu.make_async_copy(rowsA_v, hp_sh.at[src_v.at[a]],
                                      ssA).wait()
                pltpu.async_copy(wh_hbm.at[cid].at[dst_v.at[a2]], rowsA_v,
                                 gsA)
                den_drain(dsA)
                compute_ex(a2, exrA_v)
                den_scatter(a2, exrA_v, dsA)
                pltpu.make_async_copy(rowsB_v, hp_sh.at[src_v.at[b]],
                                      ssB).wait()
                pltpu.async_copy(wh_hbm.at[cid].at[dst_v.at[b2]], rowsB_v,
                                 gsB)
                den_drain(dsB)
                compute_ex(b2, exrB_v)
                den_scatter(b2, exrB_v, dsB)
            return c

        lax.fori_loop(0, n2, body, 0)
        pltpu.make_async_copy(rowsA_v, hp_sh.at[pl.ds(0, 128)], ssA).wait()
        pltpu.make_async_copy(rowsB_v, hp_sh.at[pl.ds(0, 128)], ssB).wait()
        den_drain(dsA)
        den_drain(dsB)
        plsc.subcore_barrier()

        pltpu.sync_copy(den_sh.at[pl.ds(sid * NPS, NPS)], den_v)
        base = sid * NPS

        def norm_body(b, c):
            rows_v = rowsA_v
            pltpu.async_copy(hp_sh.at[pl.ds(base + b * 128, 128)], rows_v,
                             sem).wait()
            for k in range(8):
                sl = pl.ds(b * 128 + k * 16, 16)
                d0 = den_v[sl]
                r0 = 1.0 / jnp.maximum(d0, 1e-30)
                for jj in range(16):
                    j = k * 16 + jj
                    a0 = r0[jj]
                    for q in range(HW // 16):
                        qs = pl.ds(q * 16, 16)
                        rows_v[j, qs] = rows_v[j, qs] * a0
            pltpu.sync_copy(rows_v,
                            hp_out.at[cid, pl.ds(base + b * 128, 128)])
            return c

        lax.fori_loop(0, NPS // 128, norm_body, 0)

    return sc_kernel


_sc1 = _make_sc(2)
_sc23 = _make_sc(4)


def kernel(x, edge_index, W1, a1, W2, a2, W3, a3):
    src2d = jnp.pad(edge_index[0].reshape(R_E, 128), ((0, R_P - R_E), (0, 0)))
    dst2d = jnp.pad(edge_index[1].reshape(R_E, 128), ((0, R_P - R_E), (0, 0)))
    x_p = jnp.pad(x, ((0, NP - N), (0, 0)))

    wh1, s1 = _tc1(x_p, W1, a1)
    hp1 = _sc1(src2d, dst2d, s1.reshape(-1)[:2 * N], wh1)
    wh23, s23 = _tc2(hp1, W2, W3, a2, a3)
    hp23 = _sc23(src2d, dst2d, s23.reshape(-1)[:4 * N], wh23)
    return (hp23[0, :N], hp23[0, :N], hp23[1, :N])
